# Initial kernel scaffold; baseline (speedup 1.0000x reference)
import jax
import jax.numpy as jnp
from jax.experimental import pallas as pl


def kernel(atomic_nums, conv_tensor):
    raise NotImplementedError("write your pallas kernel here")


# speedup vs baseline: 377.5519x; 377.5519x over previous
import functools

import jax
import jax.numpy as jnp
from jax import lax
from jax.experimental import pallas as pl
from jax.experimental.pallas import tpu as pltpu
from jax.experimental.pallas import tpu_sc as plsc

ROWS = 16384
COLS = 200
TABLE_SIZE = 120
LANES = 16

NUM_CORES = 2
NUM_SUBCORES = 16
NUM_WORKERS = NUM_CORES * NUM_SUBCORES
ROWS_PER_WORKER = ROWS // NUM_WORKERS
CHUNK_ROWS = 128
NUM_CHUNKS = ROWS_PER_WORKER // CHUNK_ROWS

_WINDOWS = tuple(range(0, COLS - LANES + 1, LANES)) + (COLS - LANES,)


def _tec_body(x_hbm, tab_hbm, out_hbm, tab_v, in_v, out_v):
    wid = lax.axis_index("s") * NUM_CORES + lax.axis_index("c")
    pltpu.sync_copy(tab_hbm, tab_v)

    def translate_row(r, carry):
        for c in _WINDOWS:
            idx = in_v[r, pl.ds(c, LANES)]
            out_v[r, pl.ds(c, LANES)] = plsc.load_gather(tab_v, [idx])
        return carry

    for chunk in range(NUM_CHUNKS):
        row0 = wid * ROWS_PER_WORKER + chunk * CHUNK_ROWS
        pltpu.sync_copy(x_hbm.at[pl.ds(row0, CHUNK_ROWS)], in_v)
        lax.fori_loop(0, CHUNK_ROWS, translate_row, 0)
        pltpu.sync_copy(out_v, out_hbm.at[pl.ds(row0, CHUNK_ROWS)])


@jax.jit
def kernel(atomic_nums, conv_tensor):
    mesh = plsc.VectorSubcoreMesh(
        core_axis_name="c", subcore_axis_name="s"
    )
    run = pl.kernel(
        _tec_body,
        out_type=jax.ShapeDtypeStruct((ROWS, COLS), jnp.int32),
        mesh=mesh,
        scratch_types=[
            pltpu.VMEM((TABLE_SIZE,), jnp.int32),
            pltpu.VMEM((CHUNK_ROWS, COLS), jnp.int32),
            pltpu.VMEM((CHUNK_ROWS, COLS), jnp.int32),
        ],
        compiler_params=pltpu.CompilerParams(needs_layout_passes=False),
    )
    return run(atomic_nums, conv_tensor)
